# baseline (device time: 62656 ns/iter reference)
import jax
import jax.numpy as jnp
from jax import lax
from jax.experimental import pallas as pl
from jax.experimental.pallas import tpu as pltpu

N_DEV = 32
N_STEPS = 8
TILES_PER_STEP = N_DEV // N_STEPS


def kernel(x, w_mat):
    m_total, k_my = x.shape
    k_total, n = w_mat.shape
    blk_m = m_total // N_DEV
    kc = k_total // N_STEPS

    def body(x_ref, w_hbm, out_ref, xb_ref, comm_ref, wb_ref, csem, send_sem, recv_sem):
        my = lax.axis_index("i")
        my_c = my // TILES_PER_STEP
        my_off = (my % TILES_PER_STEP) * blk_m

        def chunk_of(q):
            return (my_c + q) % N_STEPS

        barrier_sem = pltpu.get_barrier_semaphore()
        for t in range(N_DEV):
            @pl.when(t != my)
            def _sig(t=t):
                pl.semaphore_signal(
                    barrier_sem, inc=1,
                    device_id=(t,), device_id_type=pl.DeviceIdType.MESH,
                )
        pl.semaphore_wait(barrier_sem, N_DEV - 1)

        xb_ref[...] = x_ref[...].astype(jnp.bfloat16)
        comm_ref[my_c, :, pl.ds(my_off, blk_m)] = xb_ref[pl.ds(my * blk_m, blk_m), :]
        for t in range(N_DEV):
            @pl.when(t != my)
            def _send(t=t):
                pltpu.make_async_remote_copy(
                    src_ref=xb_ref.at[pl.ds(t * blk_m, blk_m), :],
                    dst_ref=comm_ref.at[my_c, :, pl.ds(my_off, blk_m)],
                    send_sem=send_sem.at[t],
                    recv_sem=recv_sem.at[my],
                    device_id=(t,),
                    device_id_type=pl.DeviceIdType.MESH,
                ).start()

        def w_copy(q, slot):
            return pltpu.make_async_copy(
                w_hbm.at[pl.ds(chunk_of(q) * kc, kc), :],
                wb_ref.at[slot],
                csem.at[slot],
            )

        w_copy(0, 0).start()
        w_copy(1, 1).start()

        for q in range(N_STEPS):
            slot = q % 2
            c = chunk_of(q)
            w_copy(q, slot).wait()

            for u in range(TILES_PER_STEP):
                t = c * TILES_PER_STEP + u

                @pl.when(t != my)
                def _wait(u=u):
                    pltpu.make_async_remote_copy(
                        src_ref=comm_ref.at[c, :, pl.ds(u * blk_m, blk_m)],
                        dst_ref=comm_ref.at[c, :, pl.ds(u * blk_m, blk_m)],
                        send_sem=send_sem.at[0],
                        recv_sem=recv_sem.at[c * TILES_PER_STEP + u],
                        device_id=(0,),
                        device_id_type=pl.DeviceIdType.MESH,
                    ).wait_recv()

            xc = comm_ref[c].astype(jnp.float32)
            part = jnp.dot(xc, wb_ref[slot], preferred_element_type=jnp.float32)
            if q == 0:
                out_ref[...] = part
            else:
                out_ref[...] += part

            if q + 2 < N_STEPS:
                w_copy(q + 2, slot).start()

        for t in range(N_DEV):
            @pl.when(t != my)
            def _drain(t=t):
                pltpu.make_async_remote_copy(
                    src_ref=xb_ref.at[pl.ds(t * blk_m, blk_m), :],
                    dst_ref=comm_ref.at[my_c, :, pl.ds(my_off, blk_m)],
                    send_sem=send_sem.at[t],
                    recv_sem=recv_sem.at[my],
                    device_id=(t,),
                    device_id_type=pl.DeviceIdType.MESH,
                ).wait_send()
        y = out_ref[...]
        out_ref[...] = y * (1.0 / (1.0 + jnp.exp(-y)))

    return pl.pallas_call(
        body,
        out_shape=jax.ShapeDtypeStruct((blk_m, n), jnp.float32),
        in_specs=[
            pl.BlockSpec(memory_space=pltpu.VMEM),
            pl.BlockSpec(memory_space=pl.ANY),
        ],
        out_specs=pl.BlockSpec(memory_space=pltpu.VMEM),
        scratch_shapes=[
            pltpu.VMEM((m_total, k_my), jnp.bfloat16),
            pltpu.VMEM((N_STEPS, blk_m, kc), jnp.bfloat16),
            pltpu.VMEM((2, kc, n), jnp.float32),
            pltpu.SemaphoreType.DMA((2,)),
            pltpu.SemaphoreType.DMA((N_DEV,)),
            pltpu.SemaphoreType.DMA((N_DEV,)),
        ],
        compiler_params=pltpu.CompilerParams(
            collective_id=0,
            vmem_limit_bytes=56 * 1024 * 1024,
        ),
    )(x, w_mat)
